# baseline (speedup 1.0000x reference)
import jax
import jax.numpy as jnp
from jax import lax
from jax.experimental import pallas as pl
from jax.experimental.pallas import tpu as pltpu
from jax.experimental.pallas import tpu_sc as plsc

B, S, D, G = 16, 4096, 256, 512
NC, NS = 2, 16
NW = NC * NS
KSC = 4
KTC = B - KSC
NT = NW // KSC // 2
CV = 128
NV = CV // 16
TH = S // NT
CH = 64
NCHUNK = TH // CH
NPAIR = NCHUNK // 2
CL = 16

_SC_PARAMS = pltpu.CompilerParams(needs_layout_passes=False)


def _sum_body(x_hbm, g_hbm, sum_out, cnt_out,
              ids_v, xbuf0, xbuf1, acc, cnt_acc, sem0, sem1):
    cid = lax.axis_index("c")
    sid = lax.axis_index("s")
    w = cid * NS + sid
    bq = w // (2 * NT)
    h = (w % (2 * NT)) // NT
    t = w % NT
    col0 = pl.multiple_of(h * CV, CV)
    tokb = pl.multiple_of((KTC + bq) * S + t * TH, TH)

    def xsrc(chunk):
        off = pl.multiple_of(tokb + chunk * CH, CH)
        return x_hbm.at[pl.ds(off, CH), pl.ds(col0, CV)]

    pltpu.async_copy(xsrc(0), xbuf0, sem0)
    pltpu.async_copy(xsrc(1), xbuf1, sem1)
    pltpu.sync_copy(g_hbm.at[pl.ds(tokb, TH)], ids_v)

    zero16 = jnp.zeros((16,), jnp.float32)

    def zrow(j, _):
        for c in range(NV):
            acc[j, pl.ds(c * 16, 16)] = zero16
        cnt_acc[pl.ds(pl.multiple_of(j * CL, CL), CL)] = zero16
        return 0

    lax.fori_loop(0, G, zrow, 0)

    def process(chunk, buf, carry):
        def group(g, carry):
            r, prev, rc = carry
            gg = pl.multiple_of(g * 16, 16)
            idvec = ids_v[pl.ds(gg, 16)]
            for k in range(16):
                i = idvec[k]
                keep = jnp.where(i != prev, 0.0, 1.0)
                r = [r[c] * keep + buf[(gg + k) % CH, pl.ds(c * 16, 16)]
                     for c in range(NV)]
                for c in range(NV):
                    acc[i, pl.ds(c * 16, 16)] = r[c]
                rc = rc * keep + 1.0
                cnt_acc[pl.ds(pl.multiple_of(i * CL, CL), CL)] = rc
                prev = i
            return r, prev, rc

        g0 = chunk * (CH // 16)
        return lax.fori_loop(g0, g0 + CH // 16, group, carry)

    carry = ([zero16] * NV, jnp.int32(-1), zero16)

    def pair(p, carry):
        c0 = p * 2
        pltpu.make_async_copy(xsrc(c0), xbuf0, sem0).wait()
        carry = process(c0, xbuf0, carry)

        @pl.when(p + 1 < NPAIR)
        def _():
            pltpu.async_copy(xsrc(c0 + 2), xbuf0, sem0)

        pltpu.make_async_copy(xsrc(c0 + 1), xbuf1, sem1).wait()
        carry = process(c0 + 1, xbuf1, carry)

        @pl.when(p + 1 < NPAIR)
        def _():
            pltpu.async_copy(xsrc(c0 + 3), xbuf1, sem1)

        return carry

    lax.fori_loop(0, NPAIR, pair, carry)

    seg0 = pl.multiple_of((bq * NT + t) * G, G)
    pltpu.sync_copy(acc, sum_out.at[pl.ds(seg0, G), pl.ds(col0, CV)])

    @pl.when(h == 0)
    def _():
        cnt0 = pl.multiple_of((bq * NT + t) * G * CL, G * CL)
        pltpu.sync_copy(cnt_acc, cnt_out.at[pl.ds(cnt0, G * CL)])


_sc_sum = pl.kernel(
    _sum_body,
    out_type=(
        jax.ShapeDtypeStruct((KSC * NT * G, D), jnp.float32),
        jax.ShapeDtypeStruct((KSC * NT * G * CL,), jnp.float32),
    ),
    mesh=plsc.VectorSubcoreMesh(core_axis_name="c", subcore_axis_name="s"),
    compiler_params=_SC_PARAMS,
    scratch_types=[
        pltpu.VMEM((TH,), jnp.int32),
        pltpu.VMEM((CH, CV), jnp.float32),
        pltpu.VMEM((CH, CV), jnp.float32),
        pltpu.VMEM((G, CV), jnp.float32),
        pltpu.VMEM((G * CL,), jnp.float32),
        pltpu.SemaphoreType.DMA,
        pltpu.SemaphoreType.DMA,
    ],
)


def _tc_onehot_body(x_ref, g_ref, w_ref, bias_ref, o_ref):
    ids = g_ref[0]
    gidx = lax.broadcasted_iota(jnp.int32, (G, S), 0)
    eq = ids == gidx
    m = jnp.where(eq, 1.0, 0.0)
    cnt = jnp.sum(m, axis=1, keepdims=True)
    seg = jax.lax.dot_general(
        m.astype(jnp.bfloat16), x_ref[0].astype(jnp.bfloat16),
        (((1,), (0,)), ((), ())),
        preferred_element_type=jnp.float32)
    mean = seg * (1.0 / jnp.maximum(cnt, 1.0))
    o_ref[0] = (
        jax.lax.dot_general(
            mean, w_ref[...], (((1,), (0,)), ((), ())),
            precision=jax.lax.Precision.HIGHEST,
            preferred_element_type=jnp.float32)
        + bias_ref[...]
    )


def _tc_onehot(x, groups3, W, bias):
    return pl.pallas_call(
        _tc_onehot_body,
        grid=(KTC,),
        in_specs=[
            pl.BlockSpec((1, S, D), lambda i: (i, 0, 0)),
            pl.BlockSpec((1, 1, S), lambda i: (i, 0, 0)),
            pl.BlockSpec((D, D), lambda i: (0, 0)),
            pl.BlockSpec((1, D), lambda i: (0, 0)),
        ],
        out_specs=pl.BlockSpec((1, G, D), lambda i: (i, 0, 0)),
        out_shape=jax.ShapeDtypeStruct((KTC, G, D), jnp.float32),
    )(x, groups3, W, bias)


def _tc_encode_body(seg_ref, cnt_ref, w_ref, bias_ref, o_ref):
    cnt = sum(cnt_ref[0, j] for j in range(NT))[:, 0:1]
    seg = sum(seg_ref[0, j] for j in range(NT))
    mean = seg * (1.0 / jnp.maximum(cnt, 1.0))
    o_ref[0] = (
        jax.lax.dot_general(
            mean, w_ref[...], (((1,), (0,)), ((), ())),
            precision=jax.lax.Precision.HIGHEST,
            preferred_element_type=jnp.float32)
        + bias_ref[...]
    )


def _tc_encode(seg, cnt3, W, bias):
    return pl.pallas_call(
        _tc_encode_body,
        grid=(KSC,),
        in_specs=[
            pl.BlockSpec((1, NT, G, D), lambda i: (i, 0, 0, 0)),
            pl.BlockSpec((1, NT, G, CL), lambda i: (i, 0, 0, 0)),
            pl.BlockSpec((D, D), lambda i: (0, 0)),
            pl.BlockSpec((1, D), lambda i: (0, 0)),
        ],
        out_specs=pl.BlockSpec((1, G, D), lambda i: (i, 0, 0)),
        out_shape=jax.ShapeDtypeStruct((KSC, G, D), jnp.float32),
    )(seg, cnt3, W, bias)


def kernel(x, groups, W, b):
    bias = b.reshape(1, D)
    xflat = x.reshape(B * S, D)
    gflat = groups.reshape(B * S)
    seg, cnt = _sc_sum(xflat, gflat)
    out_tc = _tc_onehot(x[:KTC], groups[:KTC].reshape(KTC, 1, S), W, bias)
    out_sc = _tc_encode(seg.reshape(KSC, NT, G, D),
                        cnt.reshape(KSC, NT, G, CL), W, bias)
    return jnp.concatenate([out_tc, out_sc], axis=0)


# speedup vs baseline: 1.1897x; 1.1897x over previous
import jax
import jax.numpy as jnp
from jax import lax
from jax.experimental import pallas as pl
from jax.experimental.pallas import tpu as pltpu
from jax.experimental.pallas import tpu_sc as plsc

B, S, D, G = 16, 4096, 256, 512
NC, NS = 2, 16
NW = NC * NS
KSC = 8
KTC = B - KSC
CV = 128
NV = CV // 16
TH = S // 2
CH = 64
NCHUNK = TH // CH
NPAIR = NCHUNK // 2
CL = 16

_SC_PARAMS = pltpu.CompilerParams(needs_layout_passes=False)


def _sum_body(x_hbm, g_hbm, sum_out, cnt_out,
              ids_v, xbuf0, xbuf1, acc, cnt_acc, sem0, sem1):
    cid = lax.axis_index("c")
    sid = lax.axis_index("s")
    w = cid * NS + sid
    bq = w // 4
    h = (w % 4) // 2
    t = w % 2
    col0 = pl.multiple_of(h * CV, CV)
    tokb = pl.multiple_of((KTC + bq) * S + t * TH, TH)

    def xsrc(chunk):
        off = pl.multiple_of(tokb + chunk * CH, CH)
        return x_hbm.at[pl.ds(off, CH), pl.ds(col0, CV)]

    pltpu.async_copy(xsrc(0), xbuf0, sem0)
    pltpu.async_copy(xsrc(1), xbuf1, sem1)
    pltpu.sync_copy(g_hbm.at[pl.ds(tokb, TH)], ids_v)

    zero16 = jnp.zeros((16,), jnp.float32)

    def zrow(j, _):
        for c in range(NV):
            acc[j, pl.ds(c * 16, 16)] = zero16
        cnt_acc[pl.ds(pl.multiple_of(j * CL, CL), CL)] = zero16
        return 0

    lax.fori_loop(0, G, zrow, 0)

    def process(chunk, buf, carry):
        def group(g, carry):
            r, prev, rc = carry
            gg = pl.multiple_of(g * 16, 16)
            idvec = ids_v[pl.ds(gg, 16)]
            for k in range(16):
                i = idvec[k]
                keep = jnp.where(i != prev, 0.0, 1.0)
                r = [r[c] * keep + buf[(gg + k) % CH, pl.ds(c * 16, 16)]
                     for c in range(NV)]
                for c in range(NV):
                    acc[i, pl.ds(c * 16, 16)] = r[c]
                rc = rc * keep + 1.0
                cnt_acc[pl.ds(pl.multiple_of(i * CL, CL), CL)] = rc
                prev = i
            return r, prev, rc

        g0 = chunk * (CH // 16)
        return lax.fori_loop(g0, g0 + CH // 16, group, carry)

    carry = ([zero16] * NV, jnp.int32(-1), zero16)

    def pair(p, carry):
        c0 = p * 2
        pltpu.make_async_copy(xsrc(c0), xbuf0, sem0).wait()
        carry = process(c0, xbuf0, carry)

        @pl.when(p + 1 < NPAIR)
        def _():
            pltpu.async_copy(xsrc(c0 + 2), xbuf0, sem0)

        pltpu.make_async_copy(xsrc(c0 + 1), xbuf1, sem1).wait()
        carry = process(c0 + 1, xbuf1, carry)

        @pl.when(p + 1 < NPAIR)
        def _():
            pltpu.async_copy(xsrc(c0 + 3), xbuf1, sem1)

        return carry

    lax.fori_loop(0, NPAIR, pair, carry)

    seg0 = pl.multiple_of((bq * 2 + t) * G, G)
    pltpu.sync_copy(acc, sum_out.at[pl.ds(seg0, G), pl.ds(col0, CV)])

    @pl.when(h == 0)
    def _():
        cnt0 = pl.multiple_of((bq * 2 + t) * G * CL, G * CL)
        pltpu.sync_copy(cnt_acc, cnt_out.at[pl.ds(cnt0, G * CL)])


_sc_sum = pl.kernel(
    _sum_body,
    out_type=(
        jax.ShapeDtypeStruct((KSC * 2 * G, D), jnp.float32),
        jax.ShapeDtypeStruct((KSC * 2 * G * CL,), jnp.float32),
    ),
    mesh=plsc.VectorSubcoreMesh(core_axis_name="c", subcore_axis_name="s"),
    compiler_params=_SC_PARAMS,
    scratch_types=[
        pltpu.VMEM((TH,), jnp.int32),
        pltpu.VMEM((CH, CV), jnp.float32),
        pltpu.VMEM((CH, CV), jnp.float32),
        pltpu.VMEM((G, CV), jnp.float32),
        pltpu.VMEM((G * CL,), jnp.float32),
        pltpu.SemaphoreType.DMA,
        pltpu.SemaphoreType.DMA,
    ],
)


def _tc_onehot_body(x_ref, g_ref, w_ref, bias_ref, o_ref):
    ids = g_ref[0]
    gidx = lax.broadcasted_iota(jnp.int32, (G, S), 0)
    eq = ids == gidx
    m = jnp.where(eq, 1.0, 0.0)
    cnt = jnp.sum(m, axis=1, keepdims=True)
    seg = jax.lax.dot_general(
        m.astype(jnp.bfloat16), x_ref[0].astype(jnp.bfloat16),
        (((1,), (0,)), ((), ())),
        preferred_element_type=jnp.float32)
    mean = seg * (1.0 / jnp.maximum(cnt, 1.0))
    o_ref[0] = (
        jax.lax.dot_general(
            mean, w_ref[...], (((1,), (0,)), ((), ())),
            precision=jax.lax.Precision.HIGHEST,
            preferred_element_type=jnp.float32)
        + bias_ref[...]
    )


def _tc_onehot(x, groups3, W, bias):
    return pl.pallas_call(
        _tc_onehot_body,
        grid=(KTC,),
        in_specs=[
            pl.BlockSpec((1, S, D), lambda i: (i, 0, 0)),
            pl.BlockSpec((1, 1, S), lambda i: (i, 0, 0)),
            pl.BlockSpec((D, D), lambda i: (0, 0)),
            pl.BlockSpec((1, D), lambda i: (0, 0)),
        ],
        out_specs=pl.BlockSpec((1, G, D), lambda i: (i, 0, 0)),
        out_shape=jax.ShapeDtypeStruct((KTC, G, D), jnp.float32),
    )(x, groups3, W, bias)


def _tc_encode_body(seg_ref, cnt_ref, w_ref, bias_ref, o_ref):
    cnt = (cnt_ref[0, 0] + cnt_ref[0, 1])[:, 0:1]
    seg = seg_ref[0, 0] + seg_ref[0, 1]
    mean = seg * (1.0 / jnp.maximum(cnt, 1.0))
    o_ref[0] = (
        jax.lax.dot_general(
            mean, w_ref[...], (((1,), (0,)), ((), ())),
            precision=jax.lax.Precision.HIGHEST,
            preferred_element_type=jnp.float32)
        + bias_ref[...]
    )


def _tc_encode(seg, cnt3, W, bias):
    return pl.pallas_call(
        _tc_encode_body,
        grid=(KSC,),
        in_specs=[
            pl.BlockSpec((1, 2, G, D), lambda i: (i, 0, 0, 0)),
            pl.BlockSpec((1, 2, G, CL), lambda i: (i, 0, 0, 0)),
            pl.BlockSpec((D, D), lambda i: (0, 0)),
            pl.BlockSpec((1, D), lambda i: (0, 0)),
        ],
        out_specs=pl.BlockSpec((1, G, D), lambda i: (i, 0, 0)),
        out_shape=jax.ShapeDtypeStruct((KSC, G, D), jnp.float32),
    )(seg, cnt3, W, bias)


def kernel(x, groups, W, b):
    bias = b.reshape(1, D)
    xflat = x.reshape(B * S, D)
    gflat = groups.reshape(B * S)
    seg, cnt = _sc_sum(xflat, gflat)
    out_tc = _tc_onehot(x[:KTC], groups[:KTC].reshape(KTC, 1, S), W, bias)
    out_sc = _tc_encode(seg.reshape(KSC, 2, G, D),
                        cnt.reshape(KSC, 2, G, CL), W, bias)
    return jnp.concatenate([out_tc, out_sc], axis=0)
